# baseline (speedup 1.0000x reference)
import functools

import jax
import jax.numpy as jnp
from jax import lax
from jax.experimental import pallas as pl
from jax.experimental.pallas import tpu as pltpu
from jax.experimental.pallas import tpu_sc as plsc

BATCH = 4096
SEQ = 200
EMBED_DIM = 64
NUM_CLASS = 4
VOCAB = 1000000
GW = 8

GBLK = 4096
VPAD = 245 * GBLK
VS = 92 * GBLK
VT = VPAD - VS

NUM_WORKERS = 32
ROWS_PER_W = BATCH // NUM_WORKERS
SPLIT = (96, 104)
NBUF = 4

MV_PER_W = VS // NUM_WORKERS
MBLK = 512
NMBLK = MV_PER_W // MBLK

TV_PER_W = VT // NUM_WORKERS
TCH = 2176
NTCH = TV_PER_W // TCH


def _g_body(w_ref, tT_ref, o_ref):
  o_ref[...] = lax.dot_general(w_ref[...], tT_ref[...],
                               (((1,), (0,)), ((), ())),
                               preferred_element_type=jnp.float32)


@jax.jit
def _class_table(table, W):
  return pl.pallas_call(
      _g_body,
      grid=(VT // GBLK,),
      in_specs=[
          pl.BlockSpec((NUM_CLASS, EMBED_DIM), lambda i: (0, 0)),
          pl.BlockSpec((EMBED_DIM, GBLK), lambda i: (0, i + VS // GBLK)),
      ],
      out_specs=pl.BlockSpec((NUM_CLASS, GBLK), lambda i: (0, i)),
      out_shape=jax.ShapeDtypeStruct((NUM_CLASS, VT), jnp.float32),
  )(W, table.T)


def _sc_matmul_kernel(tT_hbm, w_hbm, flat_hbm, wv, buf_v, out_v):
  wid = lax.axis_index("s") * 2 + lax.axis_index("c")
  v0 = wid * MV_PER_W
  lane = lax.iota(jnp.int32, 16)
  lane8 = lane * GW

  pltpu.sync_copy(w_hbm, wv)

  @pl.when(wid == 0)
  def _zero_row():
    out_v[pl.ds(0, 16)] = jnp.zeros((16,), jnp.float32)
    pltpu.sync_copy(out_v.at[pl.ds(0, GW)], flat_hbm.at[pl.ds(VS * GW, GW)])

  @pl.loop(0, NMBLK)
  def _blk(blk):
    col0 = v0 + blk * MBLK
    pltpu.sync_copy(tT_hbm.at[:, pl.ds(col0, MBLK)], buf_v)

    for sub in range(MBLK // 128):
      def dbody(d, accs, sub=sub):
        wc = [plsc.load_gather(wv, [jnp.full((16,), 0, jnp.int32) + (c * 64)
                                    + d]) for c in range(NUM_CLASS)]
        out = list(accs)
        for g in range(8):
          x = buf_v[d, pl.ds(sub * 128 + g * 16, 16)]
          for c in range(NUM_CLASS):
            out[g * 4 + c] = out[g * 4 + c] + x * wc[c]
        return tuple(out)

      zero = jnp.zeros((16,), jnp.float32)
      accs = lax.fori_loop(0, EMBED_DIM, dbody, (zero,) * 32)
      for g in range(8):
        base = (sub * 128 + g * 16) * GW
        for c in range(NUM_CLASS):
          plsc.store_scatter(out_v, [lane8 + (base + c)], accs[g * 4 + c])

    pltpu.sync_copy(out_v, flat_hbm.at[pl.ds(col0 * GW, MBLK * GW)])


@jax.jit
def _sc_class_table(table, W):
  mesh = plsc.VectorSubcoreMesh(core_axis_name="c", subcore_axis_name="s")
  f = pl.kernel(
      _sc_matmul_kernel,
      out_type=jax.ShapeDtypeStruct((VS * GW + GW,), jnp.float32),
      mesh=mesh,
      compiler_params=pltpu.CompilerParams(use_tc_tiling_on_sc=True,
                                           needs_layout_passes=False),
      scratch_types=[
          pltpu.VMEM((NUM_CLASS * EMBED_DIM,), jnp.float32),
          pltpu.VMEM((EMBED_DIM, MBLK), jnp.float32),
          pltpu.VMEM((MBLK * GW,), jnp.float32),
      ],
  )
  return f(table.T, W.reshape(NUM_CLASS * EMBED_DIM))


def _transpose_kernel(gt_hbm, flat_hbm, buf_v, out_v):
  wid = lax.axis_index("s") * 2 + lax.axis_index("c")
  v0 = wid * TV_PER_W
  lane = lax.iota(jnp.int32, 16)
  lane8 = lane * GW

  @pl.when(wid == 0)
  def _zero_row():
    out_v[pl.ds(0, 16)] = jnp.zeros((16,), jnp.float32)
    pltpu.sync_copy(out_v.at[pl.ds(0, GW)], flat_hbm.at[pl.ds(VT * GW, GW)])

  @pl.loop(0, NTCH)
  def _chunk(k):
    off = v0 + k * TCH
    pltpu.sync_copy(gt_hbm.at[:, pl.ds(off, TCH)], buf_v)

    @pl.loop(0, TCH // 16)
    def _group(g):
      for c in range(NUM_CLASS):
        x = buf_v[c, pl.ds(g * 16, 16)]
        plsc.store_scatter(out_v, [lane8 + (g * 128 + c)], x)

    pltpu.sync_copy(out_v, flat_hbm.at[pl.ds(off * GW, TCH * GW)])


@jax.jit
def _flat_class_table(g_t):
  mesh = plsc.VectorSubcoreMesh(core_axis_name="c", subcore_axis_name="s")
  f = pl.kernel(
      _transpose_kernel,
      out_type=jax.ShapeDtypeStruct((VT * GW + GW,), jnp.float32),
      mesh=mesh,
      compiler_params=pltpu.CompilerParams(use_tc_tiling_on_sc=True,
                                           needs_layout_passes=False),
      scratch_types=[
          pltpu.VMEM((NUM_CLASS, TCH), jnp.float32),
          pltpu.VMEM((TCH * GW,), jnp.float32),
      ],
  )
  return f(g_t)


def _make_pooling_kernel(nrows, remap):
  def _pooling_kernel(text_hbm, g_hbm, pooled_hbm, idx_v, pooled_v, sems,
                      *rows_bufs):
    wid = lax.axis_index("s") * 2 + lax.axis_index("c")
    base = wid * ROWS_PER_W

    pltpu.sync_copy(text_hbm.at[pl.ds(base * SEQ, ROWS_PER_W * SEQ)], idx_v)

    @pl.loop(0, ROWS_PER_W * SEQ // 16)
    def _clamp(j):
      idx_v[pl.ds(j * 16, 16)] = remap(idx_v[pl.ds(j * 16, 16)])

    def issue(row, slot):
      off = 0
      for n in SPLIT:
        pltpu.async_copy(g_hbm.at[idx_v.at[pl.ds(row * SEQ + off, n)]],
                         rows_bufs[slot].at[pl.ds(off, n)], sems.at[slot])
        off += n

    def drain(slot):
      pltpu.make_async_copy(g_hbm.at[pl.ds(0, SEQ)], rows_bufs[slot],
                            sems.at[slot]).wait()

    for s in range(NBUF):
      issue(s, s)

    lane = lax.iota(jnp.int32, 16)
    qrow = lax.shift_right_logical(lane, 2)
    qcol = lax.bitwise_and(lane, 3)

    @pl.loop(0, ROWS_PER_W, step=NBUF)
    def _rows(i0):
      for s in range(NBUF):
        i = i0 + s
        drain(s)

        def body(u, acc, s=s):
          x = plsc.load_gather(rows_bufs[s], [u * 4 + qrow, qcol])
          return acc + x

        acc = lax.fori_loop(0, SEQ // 4, body,
                            jnp.zeros((16,), jnp.float32), unroll=5)
        pooled_v[i] = acc

        @pl.when(i + NBUF < ROWS_PER_W)
        def _():
          issue(i + NBUF, s)

    pltpu.sync_copy(pooled_v, pooled_hbm.at[pl.ds(base, ROWS_PER_W)])

  return _pooling_kernel


def _pool(text, g_flat, nrows, remap):
  mesh = plsc.VectorSubcoreMesh(core_axis_name="c", subcore_axis_name="s")
  f = pl.kernel(
      _make_pooling_kernel(nrows, remap),
      out_type=jax.ShapeDtypeStruct((BATCH, 16), jnp.float32),
      mesh=mesh,
      compiler_params=pltpu.CompilerParams(use_tc_tiling_on_sc=False,
                                           needs_layout_passes=False),
      scratch_types=[
          pltpu.VMEM((ROWS_PER_W * SEQ,), jnp.int32),
          pltpu.VMEM((ROWS_PER_W, 16), jnp.float32),
          pltpu.SemaphoreType.DMA((NBUF,)),
      ] + [
          pltpu.VMEM((SEQ, GW), jnp.float32) for _ in range(NBUF)
      ],
  )
  return f(text.reshape(BATCH * SEQ), g_flat.reshape(nrows + 1, GW))


@jax.jit
def _pooled_a(text, flat_a):
  return _pool(text, flat_a, VS, lambda v: jnp.minimum(v, VS))


@jax.jit
def _pooled_b(text, flat_b):
  def remap(v):
    t = v - VS
    return jnp.where(t < 0, VT, t)
  return _pool(text, flat_b, VT, remap)


def _linear_body(pa_ref, pb_ref, m_ref, b_ref, o_ref):
  p = pa_ref[...] + pb_ref[...]
  o_ref[...] = jnp.dot(p, m_ref[...],
                       preferred_element_type=jnp.float32) * (1.0 / SEQ) \
      + b_ref[...]


@jax.jit
def _linear(pa, pb, b):
  fold = jnp.tile(jnp.eye(NUM_CLASS, dtype=jnp.float32), (4, 1))
  return pl.pallas_call(
      _linear_body,
      out_shape=jax.ShapeDtypeStruct((BATCH, NUM_CLASS), jnp.float32),
  )(pa, pb, fold, b.reshape(1, NUM_CLASS))


def kernel(text, lengths, table, W, b):
  del lengths
  flat_a = _sc_class_table(table, W)
  g_t = _class_table(table, W)
  flat_b = _flat_class_table(g_t)
  pa = _pooled_a(text, flat_a)
  pb = _pooled_b(text, flat_b)
  return _linear(pa, pb, b)


# speedup vs baseline: 16.3884x; 16.3884x over previous
import functools

import jax
import jax.numpy as jnp
from jax import lax
from jax.experimental import pallas as pl
from jax.experimental.pallas import tpu as pltpu
from jax.experimental.pallas import tpu_sc as plsc

BATCH = 4096
SEQ = 200
EMBED_DIM = 64
NUM_CLASS = 4
VOCAB = 1000000
GW = 8

GBLK = 4096
VPAD = 245 * GBLK
VS = 92 * GBLK
VT = VPAD - VS

NUM_WORKERS = 32
ROWS_PER_W = BATCH // NUM_WORKERS
SPLIT = (96, 104)
NBUF = 4

MV_PER_W = VS // NUM_WORKERS
MBLK = 512
NMBLK = MV_PER_W // MBLK

TV_PER_W = VT // NUM_WORKERS
TCH = 2176
NTCH = TV_PER_W // TCH


def _g_body(w_ref, tT_ref, o_ref):
  o_ref[...] = lax.dot_general(w_ref[...], tT_ref[...],
                               (((1,), (0,)), ((), ())),
                               preferred_element_type=jnp.float32)


@jax.jit
def _class_table(table, W):
  return pl.pallas_call(
      _g_body,
      grid=(VT // GBLK,),
      in_specs=[
          pl.BlockSpec((NUM_CLASS, EMBED_DIM), lambda i: (0, 0)),
          pl.BlockSpec((EMBED_DIM, GBLK), lambda i: (0, i + VS // GBLK)),
      ],
      out_specs=pl.BlockSpec((NUM_CLASS, GBLK), lambda i: (0, i)),
      out_shape=jax.ShapeDtypeStruct((NUM_CLASS, VT), jnp.float32),
  )(W, table.T)


def _sc_matmul_kernel(tT_hbm, w_hbm, flat_hbm, wv, buf_v, out_v):
  wid = lax.axis_index("s") * 2 + lax.axis_index("c")
  v0 = wid * MV_PER_W
  lane = lax.iota(jnp.int32, 16)
  lane8 = lane * GW

  pltpu.sync_copy(w_hbm, wv)

  @pl.loop(0, NMBLK)
  def _blk(blk):
    col0 = v0 + blk * MBLK
    pltpu.sync_copy(tT_hbm.at[:, pl.ds(col0, MBLK)], buf_v)

    for sub in range(MBLK // 128):
      def dbody(d, accs, sub=sub):
        wc = [plsc.load_gather(wv, [jnp.full((16,), 0, jnp.int32) + (c * 64)
                                    + d]) for c in range(NUM_CLASS)]
        out = list(accs)
        for g in range(8):
          x = buf_v[d, pl.ds(sub * 128 + g * 16, 16)]
          for c in range(NUM_CLASS):
            out[g * 4 + c] = out[g * 4 + c] + x * wc[c]
        return tuple(out)

      zero = jnp.zeros((16,), jnp.float32)
      accs = lax.fori_loop(0, EMBED_DIM, dbody, (zero,) * 32)
      for g in range(8):
        base = (sub * 128 + g * 16) * GW
        for c in range(NUM_CLASS):
          plsc.store_scatter(out_v, [lane8 + (base + c)], accs[g * 4 + c])

    pltpu.sync_copy(out_v, flat_hbm.at[pl.ds(col0 * GW, MBLK * GW)])


@jax.jit
def _sc_class_table(table, W):
  mesh = plsc.VectorSubcoreMesh(core_axis_name="c", subcore_axis_name="s")
  f = pl.kernel(
      _sc_matmul_kernel,
      out_type=jax.ShapeDtypeStruct((VS * GW,), jnp.float32),
      mesh=mesh,
      compiler_params=pltpu.CompilerParams(use_tc_tiling_on_sc=True,
                                           needs_layout_passes=False),
      scratch_types=[
          pltpu.VMEM((NUM_CLASS * EMBED_DIM,), jnp.float32),
          pltpu.VMEM((EMBED_DIM, MBLK), jnp.float32),
          pltpu.VMEM((MBLK * GW,), jnp.float32),
      ],
  )
  return f(table.T, W.reshape(NUM_CLASS * EMBED_DIM))


CPCH = VS * GW // NUM_WORKERS // 8


def _transpose_kernel(gt_hbm, flata_hbm, flat_hbm, buf_v, out_v, cp_v):
  wid = lax.axis_index("s") * 2 + lax.axis_index("c")
  v0 = wid * TV_PER_W
  lane = lax.iota(jnp.int32, 16)
  lane8 = lane * GW

  @pl.loop(0, NTCH)
  def _chunk(k):
    off = v0 + k * TCH
    pltpu.sync_copy(gt_hbm.at[:, pl.ds(off, TCH)], buf_v)

    @pl.loop(0, TCH // 16)
    def _group(g):
      for c in range(NUM_CLASS):
        x = buf_v[c, pl.ds(g * 16, 16)]
        plsc.store_scatter(out_v, [lane8 + (g * 128 + c)], x)

    pltpu.sync_copy(out_v,
                    flat_hbm.at[pl.ds((VS + off) * GW, TCH * GW)])

  @pl.loop(0, 8)
  def _copy(k):
    s = (wid * 8 + k) * CPCH
    pltpu.sync_copy(flata_hbm.at[pl.ds(s, CPCH)], cp_v)
    pltpu.sync_copy(cp_v, flat_hbm.at[pl.ds(s, CPCH)])


@jax.jit
def _flat_class_table(g_t, flat_a):
  mesh = plsc.VectorSubcoreMesh(core_axis_name="c", subcore_axis_name="s")
  f = pl.kernel(
      _transpose_kernel,
      out_type=jax.ShapeDtypeStruct((VPAD * GW,), jnp.float32),
      mesh=mesh,
      compiler_params=pltpu.CompilerParams(use_tc_tiling_on_sc=True,
                                           needs_layout_passes=False),
      scratch_types=[
          pltpu.VMEM((NUM_CLASS, TCH), jnp.float32),
          pltpu.VMEM((TCH * GW,), jnp.float32),
          pltpu.VMEM((CPCH,), jnp.float32),
      ],
  )
  return f(g_t, flat_a)


def _make_pooling_kernel():
  def _pooling_kernel(text_hbm, g_hbm, pooled_hbm, idx_v, pooled_v, sems,
                      *rows_bufs):
    wid = lax.axis_index("s") * 2 + lax.axis_index("c")
    base = wid * ROWS_PER_W

    pltpu.sync_copy(text_hbm.at[pl.ds(base * SEQ, ROWS_PER_W * SEQ)], idx_v)

    def issue(row, slot):
      off = 0
      for n in SPLIT:
        pltpu.async_copy(g_hbm.at[idx_v.at[pl.ds(row * SEQ + off, n)]],
                         rows_bufs[slot].at[pl.ds(off, n)], sems.at[slot])
        off += n

    def drain(slot):
      pltpu.make_async_copy(g_hbm.at[pl.ds(0, SEQ)], rows_bufs[slot],
                            sems.at[slot]).wait()

    for s in range(NBUF):
      issue(s, s)

    lane = lax.iota(jnp.int32, 16)
    qrow = lax.shift_right_logical(lane, 2)
    qcol = lax.bitwise_and(lane, 3)

    @pl.loop(0, ROWS_PER_W, step=NBUF)
    def _rows(i0):
      for s in range(NBUF):
        i = i0 + s
        drain(s)

        def body(u, acc, s=s):
          x = plsc.load_gather(rows_bufs[s], [u * 4 + qrow, qcol])
          return acc + x

        acc = lax.fori_loop(0, SEQ // 4, body,
                            jnp.zeros((16,), jnp.float32), unroll=5)
        pooled_v[i] = acc

        @pl.when(i + NBUF < ROWS_PER_W)
        def _():
          issue(i + NBUF, s)

    pltpu.sync_copy(pooled_v, pooled_hbm.at[pl.ds(base, ROWS_PER_W)])

  return _pooling_kernel


@jax.jit
def _pooled_sum(text, g_flat):
  mesh = plsc.VectorSubcoreMesh(core_axis_name="c", subcore_axis_name="s")
  f = pl.kernel(
      _make_pooling_kernel(),
      out_type=jax.ShapeDtypeStruct((BATCH, 16), jnp.float32),
      mesh=mesh,
      compiler_params=pltpu.CompilerParams(use_tc_tiling_on_sc=False,
                                           needs_layout_passes=False),
      scratch_types=[
          pltpu.VMEM((ROWS_PER_W * SEQ,), jnp.int32),
          pltpu.VMEM((ROWS_PER_W, 16), jnp.float32),
          pltpu.SemaphoreType.DMA((NBUF,)),
      ] + [
          pltpu.VMEM((SEQ, GW), jnp.float32) for _ in range(NBUF)
      ],
  )
  return f(text.reshape(BATCH * SEQ), g_flat.reshape(VPAD, GW))


def _linear_body(p_ref, m_ref, b_ref, o_ref):
  o_ref[...] = jnp.dot(p_ref[...], m_ref[...],
                       preferred_element_type=jnp.float32) * (1.0 / SEQ) \
      + b_ref[...]


@jax.jit
def _linear(pooled16, b):
  fold = jnp.tile(jnp.eye(NUM_CLASS, dtype=jnp.float32), (4, 1))
  return pl.pallas_call(
      _linear_body,
      out_shape=jax.ShapeDtypeStruct((BATCH, NUM_CLASS), jnp.float32),
  )(pooled16, fold, b.reshape(1, NUM_CLASS))


def kernel(text, lengths, table, W, b):
  del lengths
  flat_a = _sc_class_table(table, W)
  g_t = _class_table(table, W)
  flat = _flat_class_table(g_t, flat_a)
  pooled16 = _pooled_sum(text, flat)
  return _linear(pooled16, b)


# speedup vs baseline: 18.5620x; 1.1326x over previous
import functools

import jax
import jax.numpy as jnp
from jax import lax
from jax.experimental import pallas as pl
from jax.experimental.pallas import tpu as pltpu
from jax.experimental.pallas import tpu_sc as plsc

BATCH = 4096
SEQ = 200
EMBED_DIM = 64
NUM_CLASS = 4
VOCAB = 1000000
GW = 8

GBLK = 4096
VPAD = 245 * GBLK
VS = 92 * GBLK
VT = VPAD - VS

NUM_WORKERS = 32
ROWS_PER_W = BATCH // NUM_WORKERS
SPLIT = (96, 104)
NBUF = 4

MV_PER_W = VS // NUM_WORKERS
MBLK = 512
NMBLK = MV_PER_W // MBLK

TV_PER_W = VT // NUM_WORKERS
TCH = 2176
NTCH = TV_PER_W // TCH


def _g_body(w_ref, tT_ref, o_ref):
  o_ref[...] = lax.dot_general(w_ref[...], tT_ref[...],
                               (((1,), (0,)), ((), ())),
                               preferred_element_type=jnp.float32)


@jax.jit
def _class_table(table, W):
  return pl.pallas_call(
      _g_body,
      grid=(VT // GBLK,),
      in_specs=[
          pl.BlockSpec((NUM_CLASS, EMBED_DIM), lambda i: (0, 0)),
          pl.BlockSpec((EMBED_DIM, GBLK), lambda i: (0, i + VS // GBLK)),
      ],
      out_specs=pl.BlockSpec((NUM_CLASS, GBLK), lambda i: (0, i)),
      out_shape=jax.ShapeDtypeStruct((NUM_CLASS, VT), jnp.float32),
  )(W, table.T)


def _sc_matmul_kernel(tT_hbm, w_hbm, flat_hbm, wv, buf0, buf1, out_v,
                      isems):
  wid = lax.axis_index("s") * 2 + lax.axis_index("c")
  v0 = wid * MV_PER_W
  lane = lax.iota(jnp.int32, 16)
  lane8 = lane * GW
  bufs = (buf0, buf1)

  pltpu.sync_copy(w_hbm, wv)

  def issue(blk, s):
    pltpu.async_copy(tT_hbm.at[:, pl.ds(v0 + blk * MBLK, MBLK)], bufs[s],
                     isems.at[s])

  def drain(s):
    pltpu.make_async_copy(tT_hbm.at[:, pl.ds(0, MBLK)], bufs[s],
                          isems.at[s]).wait()

  def compute(blk, s):
    buf_v = bufs[s]
    for sub in range(MBLK // 128):
      def dbody(d, accs, sub=sub, buf_v=buf_v):
        wc = [plsc.load_gather(wv, [jnp.full((16,), 0, jnp.int32) + (c * 64)
                                    + d]) for c in range(NUM_CLASS)]
        out = list(accs)
        for g in range(8):
          x = buf_v[d, pl.ds(sub * 128 + g * 16, 16)]
          for c in range(NUM_CLASS):
            out[g * 4 + c] = out[g * 4 + c] + x * wc[c]
        return tuple(out)

      zero = jnp.zeros((16,), jnp.float32)
      accs = lax.fori_loop(0, EMBED_DIM, dbody, (zero,) * 32)
      for g in range(8):
        base = (sub * 128 + g * 16) * GW
        for c in range(NUM_CLASS):
          plsc.store_scatter(out_v, [lane8 + (base + c)], accs[g * 4 + c])

    pltpu.sync_copy(out_v,
                    flat_hbm.at[pl.ds((v0 + blk * MBLK) * GW, MBLK * GW)])

  issue(0, 0)

  @pl.loop(1, NMBLK, step=2)
  def _pair(k0):
    for j in range(2):
      blk = k0 - 1 + j
      s = j
      @pl.when(blk + 1 < NMBLK)
      def _(blk=blk, s=s):
        issue(blk + 1, 1 - s)
      drain(s)
      compute(blk, s)

  drain(0)
  compute(NMBLK - 1, 0)


@jax.jit
def _sc_class_table(table, W):
  mesh = plsc.VectorSubcoreMesh(core_axis_name="c", subcore_axis_name="s")
  f = pl.kernel(
      _sc_matmul_kernel,
      out_type=jax.ShapeDtypeStruct((VS * GW,), jnp.float32),
      mesh=mesh,
      compiler_params=pltpu.CompilerParams(use_tc_tiling_on_sc=True,
                                           needs_layout_passes=False),
      scratch_types=[
          pltpu.VMEM((NUM_CLASS * EMBED_DIM,), jnp.float32),
          pltpu.VMEM((EMBED_DIM, MBLK), jnp.float32),
          pltpu.VMEM((EMBED_DIM, MBLK), jnp.float32),
          pltpu.VMEM((MBLK * GW,), jnp.float32),
          pltpu.SemaphoreType.DMA((2,)),
      ],
  )
  return f(table.T, W.reshape(NUM_CLASS * EMBED_DIM))


NCP = 8
CPCH = VS * GW // NUM_WORKERS // NCP


def _transpose_kernel(gt_hbm, flata_hbm, flat_hbm, buf0, buf1, out0, out1,
                      cp0, cp1, isems, osems, csems, wsems):
  wid = lax.axis_index("s") * 2 + lax.axis_index("c")
  v0 = wid * TV_PER_W
  lane = lax.iota(jnp.int32, 16)
  lane8 = lane * GW
  bufs, outs, cps = (buf0, buf1), (out0, out1), (cp0, cp1)

  def cp_base(k):
    return (wid * NCP + k) * CPCH

  def issue_in(k):
    pltpu.async_copy(gt_hbm.at[:, pl.ds(v0 + k * TCH, TCH)], bufs[k % 2],
                     isems.at[k % 2])

  def wait_in(k):
    pltpu.make_async_copy(gt_hbm.at[:, pl.ds(0, TCH)], bufs[k % 2],
                          isems.at[k % 2]).wait()

  def wait_out(k):
    pltpu.make_async_copy(flata_hbm.at[pl.ds(0, TCH * GW)], outs[k % 2],
                          osems.at[k % 2]).wait()

  def issue_cp_read(k):
    pltpu.async_copy(flata_hbm.at[pl.ds(cp_base(k), CPCH)], cps[k % 2],
                     csems.at[k % 2])

  def wait_cp_read(k):
    pltpu.make_async_copy(flata_hbm.at[pl.ds(0, CPCH)], cps[k % 2],
                          csems.at[k % 2]).wait()

  def wait_cp_write(k):
    pltpu.make_async_copy(flata_hbm.at[pl.ds(0, CPCH)], cps[k % 2],
                          wsems.at[k % 2]).wait()

  issue_in(0)
  issue_cp_read(0)
  issue_in(1)
  issue_cp_read(1)

  for k in range(NTCH):
    wait_in(k)
    if k >= 2:
      wait_out(k)

    out_v = outs[k % 2]

    @pl.loop(0, TCH // 16)
    def _group(g, k=k, out_v=out_v):
      for c in range(NUM_CLASS):
        x = bufs[k % 2][c, pl.ds(g * 16, 16)]
        plsc.store_scatter(out_v, [lane8 + (g * 128 + c)], x)

    pltpu.async_copy(out_v,
                     flat_hbm.at[pl.ds((VS + v0 + k * TCH) * GW, TCH * GW)],
                     osems.at[k % 2])
    if k + 2 < NTCH:
      issue_in(k + 2)

  for k in range(NCP):
    wait_cp_read(k)
    pltpu.async_copy(cps[k % 2], flat_hbm.at[pl.ds(cp_base(k), CPCH)],
                     wsems.at[k % 2])
    if k + 2 < NCP:
      wait_cp_write(k)
      issue_cp_read(k + 2)

  for k in (NCP - 2, NCP - 1):
    wait_cp_write(k)
  for k in (NTCH - 2, NTCH - 1):
    wait_out(k)


@jax.jit
def _flat_class_table(g_t, flat_a):
  mesh = plsc.VectorSubcoreMesh(core_axis_name="c", subcore_axis_name="s")
  f = pl.kernel(
      _transpose_kernel,
      out_type=jax.ShapeDtypeStruct((VPAD * GW,), jnp.float32),
      mesh=mesh,
      compiler_params=pltpu.CompilerParams(use_tc_tiling_on_sc=True,
                                           needs_layout_passes=False),
      scratch_types=[
          pltpu.VMEM((NUM_CLASS, TCH), jnp.float32),
          pltpu.VMEM((NUM_CLASS, TCH), jnp.float32),
          pltpu.VMEM((TCH * GW,), jnp.float32),
          pltpu.VMEM((TCH * GW,), jnp.float32),
          pltpu.VMEM((CPCH,), jnp.float32),
          pltpu.VMEM((CPCH,), jnp.float32),
          pltpu.SemaphoreType.DMA((2,)),
          pltpu.SemaphoreType.DMA((2,)),
          pltpu.SemaphoreType.DMA((2,)),
          pltpu.SemaphoreType.DMA((2,)),
      ],
  )
  return f(g_t, flat_a)


def _make_pooling_kernel():
  def _pooling_kernel(text_hbm, g_hbm, pooled_hbm, idx_v, pooled_v, sems,
                      *rows_bufs):
    wid = lax.axis_index("s") * 2 + lax.axis_index("c")
    base = wid * ROWS_PER_W

    pltpu.sync_copy(text_hbm.at[pl.ds(base * SEQ, ROWS_PER_W * SEQ)], idx_v)

    def issue(row, slot):
      off = 0
      for n in SPLIT:
        pltpu.async_copy(g_hbm.at[idx_v.at[pl.ds(row * SEQ + off, n)]],
                         rows_bufs[slot].at[pl.ds(off, n)], sems.at[slot])
        off += n

    def drain(slot):
      pltpu.make_async_copy(g_hbm.at[pl.ds(0, SEQ)], rows_bufs[slot],
                            sems.at[slot]).wait()

    for s in range(NBUF):
      issue(s, s)

    lane = lax.iota(jnp.int32, 16)
    qrow = lax.shift_right_logical(lane, 2)
    qcol = lax.bitwise_and(lane, 3)

    @pl.loop(0, ROWS_PER_W, step=NBUF)
    def _rows(i0):
      for s in range(NBUF):
        i = i0 + s
        drain(s)

        def body(u, acc, s=s):
          x = plsc.load_gather(rows_bufs[s], [u * 4 + qrow, qcol])
          return acc + x

        acc = lax.fori_loop(0, SEQ // 4, body,
                            jnp.zeros((16,), jnp.float32), unroll=5)
        pooled_v[i] = acc

        @pl.when(i + NBUF < ROWS_PER_W)
        def _():
          issue(i + NBUF, s)

    pltpu.sync_copy(pooled_v, pooled_hbm.at[pl.ds(base, ROWS_PER_W)])

  return _pooling_kernel


@jax.jit
def _pooled_sum(text, g_flat):
  mesh = plsc.VectorSubcoreMesh(core_axis_name="c", subcore_axis_name="s")
  f = pl.kernel(
      _make_pooling_kernel(),
      out_type=jax.ShapeDtypeStruct((BATCH, 16), jnp.float32),
      mesh=mesh,
      compiler_params=pltpu.CompilerParams(use_tc_tiling_on_sc=False,
                                           needs_layout_passes=False),
      scratch_types=[
          pltpu.VMEM((ROWS_PER_W * SEQ,), jnp.int32),
          pltpu.VMEM((ROWS_PER_W, 16), jnp.float32),
          pltpu.SemaphoreType.DMA((NBUF,)),
      ] + [
          pltpu.VMEM((SEQ, GW), jnp.float32) for _ in range(NBUF)
      ],
  )
  return f(text.reshape(BATCH * SEQ), g_flat.reshape(VPAD, GW))


def _linear_body(p_ref, m_ref, b_ref, o_ref):
  o_ref[...] = jnp.dot(p_ref[...], m_ref[...],
                       preferred_element_type=jnp.float32) * (1.0 / SEQ) \
      + b_ref[...]


@jax.jit
def _linear(pooled16, b):
  fold = jnp.tile(jnp.eye(NUM_CLASS, dtype=jnp.float32), (4, 1))
  return pl.pallas_call(
      _linear_body,
      out_shape=jax.ShapeDtypeStruct((BATCH, NUM_CLASS), jnp.float32),
  )(pooled16, fold, b.reshape(1, NUM_CLASS))


def kernel(text, lengths, table, W, b):
  del lengths
  flat_a = _sc_class_table(table, W)
  g_t = _class_table(table, W)
  flat = _flat_class_table(g_t, flat_a)
  pooled16 = _pooled_sum(text, flat)
  return _linear(pooled16, b)


# speedup vs baseline: 19.0702x; 1.0274x over previous
import functools

import jax
import jax.numpy as jnp
from jax import lax
from jax.experimental import pallas as pl
from jax.experimental.pallas import tpu as pltpu
from jax.experimental.pallas import tpu_sc as plsc

BATCH = 4096
SEQ = 200
EMBED_DIM = 64
NUM_CLASS = 4
VOCAB = 1000000
GW = 8

GBLK = 4096
VPAD = 245 * GBLK
VS = 92 * GBLK
VT = VPAD - VS

NUM_WORKERS = 32
ROWS_PER_W = BATCH // NUM_WORKERS
SPLIT = (96, 104)
NBUF = 8

MV_PER_W = VS // NUM_WORKERS
MBLK = 512
NMBLK = MV_PER_W // MBLK

TV_PER_W = VT // NUM_WORKERS
TCH = 2176
NTCH = TV_PER_W // TCH


def _g_body(w_ref, tT_ref, o_ref):
  o_ref[...] = lax.dot_general(w_ref[...], tT_ref[...],
                               (((1,), (0,)), ((), ())),
                               preferred_element_type=jnp.float32)


@jax.jit
def _class_table(table, W):
  return pl.pallas_call(
      _g_body,
      grid=(VT // GBLK,),
      in_specs=[
          pl.BlockSpec((NUM_CLASS, EMBED_DIM), lambda i: (0, 0)),
          pl.BlockSpec((EMBED_DIM, GBLK), lambda i: (0, i + VS // GBLK)),
      ],
      out_specs=pl.BlockSpec((NUM_CLASS, GBLK), lambda i: (0, i)),
      out_shape=jax.ShapeDtypeStruct((NUM_CLASS, VT), jnp.float32),
  )(W, table.T)


def _sc_matmul_kernel(tT_hbm, w_hbm, flat_hbm, wv, buf0, buf1, out_v,
                      isems):
  wid = lax.axis_index("s") * 2 + lax.axis_index("c")
  v0 = wid * MV_PER_W
  lane = lax.iota(jnp.int32, 16)
  lane8 = lane * GW
  bufs = (buf0, buf1)

  pltpu.sync_copy(w_hbm, wv)

  def issue(blk, s):
    pltpu.async_copy(tT_hbm.at[:, pl.ds(v0 + blk * MBLK, MBLK)], bufs[s],
                     isems.at[s])

  def drain(s):
    pltpu.make_async_copy(tT_hbm.at[:, pl.ds(0, MBLK)], bufs[s],
                          isems.at[s]).wait()

  def compute(blk, s):
    buf_v = bufs[s]
    for sub in range(MBLK // 128):
      def dbody(d, accs, sub=sub, buf_v=buf_v):
        wc = [plsc.load_gather(wv, [jnp.full((16,), 0, jnp.int32) + (c * 64)
                                    + d]) for c in range(NUM_CLASS)]
        out = list(accs)
        for g in range(8):
          x = buf_v[d, pl.ds(sub * 128 + g * 16, 16)]
          for c in range(NUM_CLASS):
            out[g * 4 + c] = out[g * 4 + c] + x * wc[c]
        return tuple(out)

      zero = jnp.zeros((16,), jnp.float32)
      accs = lax.fori_loop(0, EMBED_DIM, dbody, (zero,) * 32)
      for g in range(8):
        base = (sub * 128 + g * 16) * GW
        for c in range(NUM_CLASS):
          plsc.store_scatter(out_v, [lane8 + (base + c)], accs[g * 4 + c])

    pltpu.sync_copy(out_v,
                    flat_hbm.at[pl.ds((v0 + blk * MBLK) * GW, MBLK * GW)])

  issue(0, 0)

  @pl.loop(1, NMBLK, step=2)
  def _pair(k0):
    for j in range(2):
      blk = k0 - 1 + j
      s = j
      @pl.when(blk + 1 < NMBLK)
      def _(blk=blk, s=s):
        issue(blk + 1, 1 - s)
      drain(s)
      compute(blk, s)

  drain(0)
  compute(NMBLK - 1, 0)


@jax.jit
def _sc_class_table(table, W):
  mesh = plsc.VectorSubcoreMesh(core_axis_name="c", subcore_axis_name="s")
  f = pl.kernel(
      _sc_matmul_kernel,
      out_type=jax.ShapeDtypeStruct((VS * GW,), jnp.float32),
      mesh=mesh,
      compiler_params=pltpu.CompilerParams(use_tc_tiling_on_sc=True,
                                           needs_layout_passes=False),
      scratch_types=[
          pltpu.VMEM((NUM_CLASS * EMBED_DIM,), jnp.float32),
          pltpu.VMEM((EMBED_DIM, MBLK), jnp.float32),
          pltpu.VMEM((EMBED_DIM, MBLK), jnp.float32),
          pltpu.VMEM((MBLK * GW,), jnp.float32),
          pltpu.SemaphoreType.DMA((2,)),
      ],
  )
  return f(table.T, W.reshape(NUM_CLASS * EMBED_DIM))


NCP = 8
CPCH = VS * GW // NUM_WORKERS // NCP


def _transpose_kernel(gt_hbm, flata_hbm, flat_hbm, buf0, buf1, out0, out1,
                      cp0, cp1, isems, osems, csems, wsems):
  wid = lax.axis_index("s") * 2 + lax.axis_index("c")
  v0 = wid * TV_PER_W
  lane = lax.iota(jnp.int32, 16)
  lane8 = lane * GW
  bufs, outs, cps = (buf0, buf1), (out0, out1), (cp0, cp1)

  def cp_base(k):
    return (wid * NCP + k) * CPCH

  def issue_in(k):
    pltpu.async_copy(gt_hbm.at[:, pl.ds(v0 + k * TCH, TCH)], bufs[k % 2],
                     isems.at[k % 2])

  def wait_in(k):
    pltpu.make_async_copy(gt_hbm.at[:, pl.ds(0, TCH)], bufs[k % 2],
                          isems.at[k % 2]).wait()

  def wait_out(k):
    pltpu.make_async_copy(flata_hbm.at[pl.ds(0, TCH * GW)], outs[k % 2],
                          osems.at[k % 2]).wait()

  def issue_cp_read(k):
    pltpu.async_copy(flata_hbm.at[pl.ds(cp_base(k), CPCH)], cps[k % 2],
                     csems.at[k % 2])

  def wait_cp_read(k):
    pltpu.make_async_copy(flata_hbm.at[pl.ds(0, CPCH)], cps[k % 2],
                          csems.at[k % 2]).wait()

  def wait_cp_write(k):
    pltpu.make_async_copy(flata_hbm.at[pl.ds(0, CPCH)], cps[k % 2],
                          wsems.at[k % 2]).wait()

  issue_in(0)
  issue_cp_read(0)
  issue_in(1)
  issue_cp_read(1)

  for k in range(NTCH):
    wait_in(k)
    if k >= 2:
      wait_out(k)

    out_v = outs[k % 2]

    @pl.loop(0, TCH // 16)
    def _group(g, k=k, out_v=out_v):
      for c in range(NUM_CLASS):
        x = bufs[k % 2][c, pl.ds(g * 16, 16)]
        plsc.store_scatter(out_v, [lane8 + (g * 128 + c)], x)

    pltpu.async_copy(out_v,
                     flat_hbm.at[pl.ds((VS + v0 + k * TCH) * GW, TCH * GW)],
                     osems.at[k % 2])
    if k + 2 < NTCH:
      issue_in(k + 2)

  for k in range(NCP):
    wait_cp_read(k)
    pltpu.async_copy(cps[k % 2], flat_hbm.at[pl.ds(cp_base(k), CPCH)],
                     wsems.at[k % 2])
    if k + 2 < NCP:
      wait_cp_write(k)
      issue_cp_read(k + 2)

  for k in (NCP - 2, NCP - 1):
    wait_cp_write(k)
  for k in (NTCH - 2, NTCH - 1):
    wait_out(k)


@jax.jit
def _flat_class_table(g_t, flat_a):
  mesh = plsc.VectorSubcoreMesh(core_axis_name="c", subcore_axis_name="s")
  f = pl.kernel(
      _transpose_kernel,
      out_type=jax.ShapeDtypeStruct((VPAD * GW,), jnp.float32),
      mesh=mesh,
      compiler_params=pltpu.CompilerParams(use_tc_tiling_on_sc=True,
                                           needs_layout_passes=False),
      scratch_types=[
          pltpu.VMEM((NUM_CLASS, TCH), jnp.float32),
          pltpu.VMEM((NUM_CLASS, TCH), jnp.float32),
          pltpu.VMEM((TCH * GW,), jnp.float32),
          pltpu.VMEM((TCH * GW,), jnp.float32),
          pltpu.VMEM((CPCH,), jnp.float32),
          pltpu.VMEM((CPCH,), jnp.float32),
          pltpu.SemaphoreType.DMA((2,)),
          pltpu.SemaphoreType.DMA((2,)),
          pltpu.SemaphoreType.DMA((2,)),
          pltpu.SemaphoreType.DMA((2,)),
      ],
  )
  return f(g_t, flat_a)


def _make_pooling_kernel():
  def _pooling_kernel(text_hbm, g_hbm, pooled_hbm, idx_v, pooled_v, sems,
                      *rows_bufs):
    wid = lax.axis_index("s") * 2 + lax.axis_index("c")
    base = wid * ROWS_PER_W

    pltpu.sync_copy(text_hbm.at[pl.ds(base * SEQ, ROWS_PER_W * SEQ)], idx_v)

    def issue(row, slot):
      off = 0
      for n in SPLIT:
        pltpu.async_copy(g_hbm.at[idx_v.at[pl.ds(row * SEQ + off, n)]],
                         rows_bufs[slot].at[pl.ds(off, n)], sems.at[slot])
        off += n

    def drain(slot):
      pltpu.make_async_copy(g_hbm.at[pl.ds(0, SEQ)], rows_bufs[slot],
                            sems.at[slot]).wait()

    for s in range(NBUF):
      issue(s, s)

    lane = lax.iota(jnp.int32, 16)
    qrow = lax.shift_right_logical(lane, 2)
    qcol = lax.bitwise_and(lane, 3)

    @pl.loop(0, ROWS_PER_W, step=NBUF)
    def _rows(i0):
      for s in range(NBUF):
        i = i0 + s
        drain(s)

        def body(u, acc, s=s):
          x = plsc.load_gather(rows_bufs[s], [u * 4 + qrow, qcol])
          return acc + x

        acc = lax.fori_loop(0, SEQ // 4, body,
                            jnp.zeros((16,), jnp.float32), unroll=10)
        pooled_v[i] = acc

        @pl.when(i + NBUF < ROWS_PER_W)
        def _():
          issue(i + NBUF, s)

    pltpu.sync_copy(pooled_v, pooled_hbm.at[pl.ds(base, ROWS_PER_W)])

  return _pooling_kernel


@jax.jit
def _pooled_sum(text, g_flat):
  mesh = plsc.VectorSubcoreMesh(core_axis_name="c", subcore_axis_name="s")
  f = pl.kernel(
      _make_pooling_kernel(),
      out_type=jax.ShapeDtypeStruct((BATCH, 16), jnp.float32),
      mesh=mesh,
      compiler_params=pltpu.CompilerParams(use_tc_tiling_on_sc=False,
                                           needs_layout_passes=False),
      scratch_types=[
          pltpu.VMEM((ROWS_PER_W * SEQ,), jnp.int32),
          pltpu.VMEM((ROWS_PER_W, 16), jnp.float32),
          pltpu.SemaphoreType.DMA((NBUF,)),
      ] + [
          pltpu.VMEM((SEQ, GW), jnp.float32) for _ in range(NBUF)
      ],
  )
  return f(text.reshape(BATCH * SEQ), g_flat.reshape(VPAD, GW))


def _linear_body(p_ref, m_ref, b_ref, o_ref):
  o_ref[...] = jnp.dot(p_ref[...], m_ref[...],
                       preferred_element_type=jnp.float32) * (1.0 / SEQ) \
      + b_ref[...]


@jax.jit
def _linear(pooled16, b):
  fold = jnp.tile(jnp.eye(NUM_CLASS, dtype=jnp.float32), (4, 1))
  return pl.pallas_call(
      _linear_body,
      out_shape=jax.ShapeDtypeStruct((BATCH, NUM_CLASS), jnp.float32),
  )(pooled16, fold, b.reshape(1, NUM_CLASS))


def kernel(text, lengths, table, W, b):
  del lengths
  flat_a = _sc_class_table(table, W)
  g_t = _class_table(table, W)
  flat = _flat_class_table(g_t, flat_a)
  pooled16 = _pooled_sum(text, flat)
  return _linear(pooled16, b)
